# Initial kernel scaffold; baseline (speedup 1.0000x reference)
import jax
import jax.numpy as jnp
from jax.experimental import pallas as pl


def kernel(x, edge_index, W1, b1, W2, b2):
    raise NotImplementedError("write your pallas kernel here")


# speedup vs baseline: 91.2715x; 91.2715x over previous
import functools

import jax
import jax.numpy as jnp
from jax import lax
from jax.experimental import pallas as pl
from jax.experimental.pallas import tpu as pltpu
from jax.experimental.pallas import tpu_sc as plsc

N_NODES = 10000
NPAD = 10240
D_IN = 128
D_HID = 128
N_CLS = 64
E_TOT = 320000
CH = 80
NT = 16
NC = 2

@functools.cache
def _mesh():
    return plsc.VectorSubcoreMesh(
        core_axis_name="c", subcore_axis_name="s",
        num_cores=NC, num_subcores=NT)


def _i32(v):
    return lax.convert_element_type(v, jnp.int32)


def _loop(n, body):
    lax.fori_loop(jnp.int32(0), jnp.int32(n), lambda i, _: (body(i), None)[1],
                  None)


def _fill(vref, n, value):
    def body(i):
        vref[pl.ds(i * jnp.int32(16), 16)] = jnp.full((16,), value,
                                                      jnp.float32)
    _loop(n // 16, body)


def _deg_body(dst_hbm, deg_out, acc, idx_v, ones_v, stage_v):
    c = _i32(lax.axis_index("c"))
    s = _i32(lax.axis_index("s"))
    zpt = NPAD // NT
    _fill(stage_v, zpt, 0.0)
    _fill(ones_v, CH, 1.0)
    tbase = s * jnp.int32(zpt)
    pltpu.sync_copy(stage_v, acc.at[pl.ds(tbase, zpt)])
    plsc.subcore_barrier()
    ept = E_TOT // (NC * NT)
    base = (c * jnp.int32(NT) + s) * jnp.int32(ept)

    def edge_chunk(j):
        off = base + j * jnp.int32(CH)
        pltpu.sync_copy(dst_hbm.at[pl.ds(off, CH)], idx_v)
        pltpu.sync_copy(ones_v, acc.at[idx_v], add=True)

    _loop(ept // CH, edge_chunk)

    plsc.subcore_barrier()
    pltpu.sync_copy(acc.at[pl.ds(tbase, zpt)], stage_v)
    pltpu.sync_copy(
        stage_v, deg_out.at[pl.ds(c * jnp.int32(NPAD) + tbase, zpt)])


@functools.cache
def _deg_call():
    return pl.kernel(
        _deg_body,
        out_type=jax.ShapeDtypeStruct((NC * NPAD,), jnp.float32),
        mesh=_mesh(),
        scratch_types=[
            pltpu.VMEM_SHARED((NPAD,), jnp.float32),
            pltpu.VMEM((CH,), jnp.int32),
            pltpu.VMEM((CH,), jnp.float32),
            pltpu.VMEM((NPAD // NT,), jnp.float32),
        ],
        compiler_params=pltpu.CompilerParams(use_tc_tiling_on_sc=False),
    )


def _agg_body(dh, src_hbm, dst_hbm, ys_hbm, out_hbm,
              acc, sidx_v, didx_v, rows_v, gsem):
    c = _i32(lax.axis_index("c"))
    s = _i32(lax.axis_index("s"))
    rpt = NPAD // NT
    def zrow(r):
        for k in range(dh // 16):
            rows_v[r, pl.ds(k * 16, 16)] = jnp.zeros((16,), jnp.float32)

    _loop(CH, zrow)

    tbase = s * jnp.int32(rpt)
    for t in range(rpt // CH):
        pltpu.sync_copy(rows_v, acc.at[pl.ds(tbase + jnp.int32(t * CH), CH)])
    plsc.subcore_barrier()

    ept = E_TOT // NT
    base = s * jnp.int32(ept)
    coff = c * jnp.int32(NPAD)

    def edge_chunk(j):
        off = base + j * jnp.int32(CH)
        pltpu.sync_copy(src_hbm.at[pl.ds(off, CH)], sidx_v)
        for k in range(CH // 16):
            sidx_v[pl.ds(k * 16, 16)] = sidx_v[pl.ds(k * 16, 16)] + coff
        pltpu.async_copy(ys_hbm.at[sidx_v], rows_v, gsem).wait()
        pltpu.sync_copy(dst_hbm.at[pl.ds(off, CH)], didx_v)
        pltpu.sync_copy(rows_v, acc.at[didx_v], add=True)

    _loop(ept // CH, edge_chunk)

    plsc.subcore_barrier()
    obase = c * jnp.int32(NPAD) + tbase
    for t in range(rpt // CH):
        pltpu.sync_copy(acc.at[pl.ds(tbase + jnp.int32(t * CH), CH)], rows_v)
        pltpu.sync_copy(rows_v, out_hbm.at[pl.ds(obase + jnp.int32(t * CH), CH)])


@functools.cache
def _make_agg(dh):
    return pl.kernel(
        functools.partial(_agg_body, dh),
        out_type=jax.ShapeDtypeStruct((NC * NPAD, dh), jnp.float32),
        mesh=_mesh(),
        scratch_types=[
            pltpu.VMEM_SHARED((NPAD, dh), jnp.float32),
            pltpu.VMEM((CH,), jnp.int32),
            pltpu.VMEM((CH,), jnp.int32),
            pltpu.VMEM((CH, dh), jnp.float32),
            pltpu.SemaphoreType.DMA,
        ],
        compiler_params=pltpu.CompilerParams(use_tc_tiling_on_sc=False),
    )


RB = 1024


def _tc1_body(degp_ref, x_ref, w_ref, ys_ref, dis_ref):
    deg = degp_ref[0] + degp_ref[1] + 1.0
    dis = lax.rsqrt(deg)
    dis_ref[...] = dis
    ys_ref[0] = jnp.dot(x_ref[...], w_ref[0],
                        preferred_element_type=jnp.float32) * dis


def _tc1_call(degp, x_p, w1h):
    return pl.pallas_call(
        _tc1_body,
        grid=(NPAD // RB, 2),
        in_specs=[
            pl.BlockSpec((2, RB, 1), lambda r, h: (0, r, 0)),
            pl.BlockSpec((RB, D_IN), lambda r, h: (r, 0)),
            pl.BlockSpec((1, D_IN, 64), lambda r, h: (h, 0, 0)),
        ],
        out_specs=[
            pl.BlockSpec((1, RB, 64), lambda r, h: (h, r, 0)),
            pl.BlockSpec((RB, 1), lambda r, h: (r, 0)),
        ],
        out_shape=[
            jax.ShapeDtypeStruct((2, NPAD, 64), jnp.float32),
            jax.ShapeDtypeStruct((NPAD, 1), jnp.float32),
        ],
    )(degp, x_p, w1h)


def _tc2_body(agg_ref, ys1_ref, dis_ref, b1_ref, w2_ref, ys2_ref):
    dis = dis_ref[...]
    h0 = jnp.maximum(dis * (agg_ref[0] + ys1_ref[0]) + b1_ref[0], 0.0)
    h1 = jnp.maximum(dis * (agg_ref[1] + ys1_ref[1]) + b1_ref[1], 0.0)
    acc = (jnp.dot(h0, w2_ref[0, 0], preferred_element_type=jnp.float32)
           + jnp.dot(h1, w2_ref[1, 0], preferred_element_type=jnp.float32))
    ys2_ref[0] = acc * dis


def _tc2_call(agg1, ys1, dis, b1r, w2q):
    return pl.pallas_call(
        _tc2_body,
        grid=(NPAD // RB, 2),
        in_specs=[
            pl.BlockSpec((2, RB, 64), lambda r, h: (0, r, 0)),
            pl.BlockSpec((2, RB, 64), lambda r, h: (0, r, 0)),
            pl.BlockSpec((RB, 1), lambda r, h: (r, 0)),
            pl.BlockSpec((2, 1, 64), lambda r, h: (0, 0, 0)),
            pl.BlockSpec((2, 1, 64, 32), lambda r, h: (0, h, 0, 0)),
        ],
        out_specs=pl.BlockSpec((1, RB, 32), lambda r, h: (h, r, 0)),
        out_shape=jax.ShapeDtypeStruct((2, NPAD, 32), jnp.float32),
    )(agg1, ys1, dis, b1r, w2q)


def _tc3_body(agg2_ref, ys2_ref, dis_ref, b2_ref, out_ref):
    dis = dis_ref[...]
    o0 = dis * (agg2_ref[0] + ys2_ref[0]) + b2_ref[0]
    o1 = dis * (agg2_ref[1] + ys2_ref[1]) + b2_ref[1]
    out_ref[...] = jnp.concatenate([o0, o1], axis=1)


def _tc3_call(agg2, ys2, dis, b2r):
    return pl.pallas_call(
        _tc3_body,
        grid=(NPAD // RB,),
        in_specs=[
            pl.BlockSpec((2, RB, 32), lambda r: (0, r, 0)),
            pl.BlockSpec((2, RB, 32), lambda r: (0, r, 0)),
            pl.BlockSpec((RB, 1), lambda r: (r, 0)),
            pl.BlockSpec((2, 1, 32), lambda r: (0, 0, 0)),
        ],
        out_specs=pl.BlockSpec((RB, N_CLS), lambda r: (r, 0)),
        out_shape=jax.ShapeDtypeStruct((NPAD, N_CLS), jnp.float32),
    )(agg2, ys2, dis, b2r)


def kernel(x, edge_index, W1, b1, W2, b2):
    with jax.enable_x64(False):
        return _kernel_32(x, edge_index, W1, b1, W2, b2)


def _kernel_32(x, edge_index, W1, b1, W2, b2):
    src = edge_index[0].astype(jnp.int32)
    dst = edge_index[1].astype(jnp.int32)
    x = x.astype(jnp.float32)
    W1 = W1.astype(jnp.float32)
    W2 = W2.astype(jnp.float32)
    b1 = b1.astype(jnp.float32)
    b2 = b2.astype(jnp.float32)
    x_p = jnp.zeros((NPAD, D_IN), jnp.float32).at[:N_NODES].set(x)
    w1h = jnp.stack([W1[:, :64], W1[:, 64:]])
    w2r = W2.reshape(2, 64, N_CLS)
    w2q = jnp.stack([w2r[:, :, :32], w2r[:, :, 32:]], axis=1)
    b1r = b1.reshape(2, 1, 64)
    b2r = b2.reshape(2, 1, 32)

    degp = _deg_call()(dst).reshape(2, NPAD, 1)
    ys1, dis = _tc1_call(degp, x_p, w1h)
    agg1 = _make_agg(64)(src, dst, ys1.reshape(NC * NPAD, 64))
    ys2 = _tc2_call(agg1.reshape(2, NPAD, 64), ys1, dis, b1r, w2q)
    agg2 = _make_agg(32)(src, dst, ys2.reshape(NC * NPAD, 32))
    out = _tc3_call(agg2.reshape(2, NPAD, 32), ys2, dis, b2r)
    return out[:N_NODES]


# speedup vs baseline: 298.3666x; 3.2690x over previous
import functools

import jax
import jax.numpy as jnp
from jax import lax
from jax.experimental import pallas as pl
from jax.experimental.pallas import tpu as pltpu
from jax.experimental.pallas import tpu_sc as plsc

N_NODES = 10000
NPAD = 10240
D_IN = 128
D_HID = 128
N_CLS = 64
E_TOT = 320000
CH = 80
NT = 16
NC = 2

@functools.cache
def _mesh():
    return plsc.VectorSubcoreMesh(
        core_axis_name="c", subcore_axis_name="s",
        num_cores=NC, num_subcores=NT)


def _i32(v):
    return lax.convert_element_type(v, jnp.int32)


def _loop(n, body):
    lax.fori_loop(jnp.int32(0), jnp.int32(n), lambda i, _: (body(i), None)[1],
                  None)


def _fill(vref, n, value):
    def body(i):
        vref[pl.ds(i * jnp.int32(16), 16)] = jnp.full((16,), value,
                                                      jnp.float32)
    _loop(n // 16, body)


DEG_CPT = E_TOT // (NC * NT) // CH


def _deg_body(dst_hbm, deg_out, acc, idx_all, ones_v, stage_v, sem):
    c = _i32(lax.axis_index("c"))
    s = _i32(lax.axis_index("s"))
    zpt = NPAD // NT
    _fill(stage_v, zpt, 0.0)
    _fill(ones_v, CH, 1.0)
    tbase = s * jnp.int32(zpt)
    pltpu.sync_copy(stage_v, acc.at[pl.ds(tbase, zpt)])
    pltpu.sync_copy(dst_hbm.at[c * jnp.int32(NT) + s], idx_all)
    plsc.subcore_barrier()

    def fire(j):
        pltpu.async_copy(ones_v, acc.at[idx_all.at[j]], sem, add=True)

    _loop(DEG_CPT, fire)

    def drain(j):
        pltpu.make_async_copy(ones_v, acc.at[idx_all.at[j]], sem).wait()

    _loop(DEG_CPT, drain)

    plsc.subcore_barrier()
    pltpu.sync_copy(acc.at[pl.ds(tbase, zpt)], stage_v)
    pltpu.sync_copy(
        stage_v, deg_out.at[pl.ds(c * jnp.int32(NPAD) + tbase, zpt)])


@functools.cache
def _deg_call():
    return pl.kernel(
        _deg_body,
        out_type=jax.ShapeDtypeStruct((NC * NPAD,), jnp.float32),
        mesh=_mesh(),
        scratch_types=[
            pltpu.VMEM_SHARED((NPAD,), jnp.float32),
            pltpu.VMEM((DEG_CPT, CH), jnp.int32),
            pltpu.VMEM((CH,), jnp.float32),
            pltpu.VMEM((NPAD // NT,), jnp.float32),
            pltpu.SemaphoreType.DMA,
        ],
        compiler_params=pltpu.CompilerParams(use_tc_tiling_on_sc=False),
    )


AGG_CPT = E_TOT // NT // CH
AGG_R = 5
AGG_NGRP = AGG_CPT // AGG_R


def _agg_body(dh, src_hbm, dst_hbm, ys_hbm, out_hbm,
              acc, sidx_all, didx_all, rows, gsems, ssems):
    c = _i32(lax.axis_index("c"))
    s = _i32(lax.axis_index("s"))
    rpt = NPAD // NT
    def zrow(r):
        for k in range(dh // 16):
            rows[0, r, pl.ds(k * 16, 16)] = jnp.zeros((16,), jnp.float32)

    _loop(CH, zrow)
    tbase = s * jnp.int32(rpt)
    for t in range(rpt // CH):
        pltpu.sync_copy(rows.at[0],
                        acc.at[pl.ds(tbase + jnp.int32(t * CH), CH)])
    pltpu.sync_copy(src_hbm.at[s], sidx_all)
    pltpu.sync_copy(dst_hbm.at[s], didx_all)
    coff = c * jnp.int32(NPAD)

    def arow(r):
        for k in range(CH // 16):
            sidx_all[r, pl.ds(k * 16, 16)] = (
                sidx_all[r, pl.ds(k * 16, 16)] + coff)

    _loop(AGG_CPT, arow)
    plsc.subcore_barrier()

    def gather(j, b):
        pltpu.async_copy(ys_hbm.at[sidx_all.at[j]], rows.at[b], gsems.at[b])

    def gather_wait(j, b):
        pltpu.make_async_copy(
            ys_hbm.at[sidx_all.at[j]], rows.at[b], gsems.at[b]).wait()

    def scatter(j, b):
        pltpu.async_copy(rows.at[b], acc.at[didx_all.at[j]], ssems.at[b],
                         add=True)

    def scatter_wait(j, b):
        pltpu.make_async_copy(
            rows.at[b], acc.at[didx_all.at[j]], ssems.at[b]).wait()

    for b in range(AGG_R):
        gather(jnp.int32(b), b)

    def grp(t):
        jb = t * jnp.int32(AGG_R)
        for b in range(AGG_R):
            gather_wait(jb + jnp.int32(b), b)
            scatter(jb + jnp.int32(b), b)
        for b in range(AGG_R):
            scatter_wait(jb + jnp.int32(b), b)
            gather(jb + jnp.int32(AGG_R + b), b)

    _loop(AGG_NGRP - 1, grp)

    jb = jnp.int32((AGG_NGRP - 1) * AGG_R)
    for b in range(AGG_R):
        gather_wait(jb + jnp.int32(b), b)
        scatter(jb + jnp.int32(b), b)
    for b in range(AGG_R):
        scatter_wait(jb + jnp.int32(b), b)

    plsc.subcore_barrier()
    obase = c * jnp.int32(NPAD) + tbase
    for t in range(rpt // CH):
        pltpu.sync_copy(acc.at[pl.ds(tbase + jnp.int32(t * CH), CH)],
                        rows.at[0])
        pltpu.sync_copy(rows.at[0],
                        out_hbm.at[pl.ds(obase + jnp.int32(t * CH), CH)])


@functools.cache
def _make_agg(dh):
    return pl.kernel(
        functools.partial(_agg_body, dh),
        out_type=jax.ShapeDtypeStruct((NC * NPAD, dh), jnp.float32),
        mesh=_mesh(),
        scratch_types=[
            pltpu.VMEM_SHARED((NPAD, dh), jnp.float32),
            pltpu.VMEM((AGG_CPT, CH), jnp.int32),
            pltpu.VMEM((AGG_CPT, CH), jnp.int32),
            pltpu.VMEM((AGG_R, CH, dh), jnp.float32),
            pltpu.SemaphoreType.DMA((AGG_R,)),
            pltpu.SemaphoreType.DMA((AGG_R,)),
        ],
        compiler_params=pltpu.CompilerParams(use_tc_tiling_on_sc=False),
    )


RB = 1024


def _tc1_body(degp_ref, x_ref, w_ref, ys_ref, dis_ref):
    deg = degp_ref[0] + degp_ref[1] + 1.0
    dis = lax.rsqrt(deg)
    dis_ref[...] = dis
    ys_ref[0] = jnp.dot(x_ref[...], w_ref[0],
                        preferred_element_type=jnp.float32) * dis


def _tc1_call(degp, x_p, w1h):
    return pl.pallas_call(
        _tc1_body,
        grid=(NPAD // RB, 2),
        in_specs=[
            pl.BlockSpec((2, RB, 1), lambda r, h: (0, r, 0)),
            pl.BlockSpec((RB, D_IN), lambda r, h: (r, 0)),
            pl.BlockSpec((1, D_IN, 64), lambda r, h: (h, 0, 0)),
        ],
        out_specs=[
            pl.BlockSpec((1, RB, 64), lambda r, h: (h, r, 0)),
            pl.BlockSpec((RB, 1), lambda r, h: (r, 0)),
        ],
        out_shape=[
            jax.ShapeDtypeStruct((2, NPAD, 64), jnp.float32),
            jax.ShapeDtypeStruct((NPAD, 1), jnp.float32),
        ],
    )(degp, x_p, w1h)


def _tc2_body(agg_ref, ys1_ref, dis_ref, b1_ref, w2_ref, ys2_ref):
    dis = dis_ref[...]
    h0 = jnp.maximum(dis * (agg_ref[0] + ys1_ref[0]) + b1_ref[0], 0.0)
    h1 = jnp.maximum(dis * (agg_ref[1] + ys1_ref[1]) + b1_ref[1], 0.0)
    acc = (jnp.dot(h0, w2_ref[0, 0], preferred_element_type=jnp.float32)
           + jnp.dot(h1, w2_ref[1, 0], preferred_element_type=jnp.float32))
    ys2_ref[0] = acc * dis


def _tc2_call(agg1, ys1, dis, b1r, w2q):
    return pl.pallas_call(
        _tc2_body,
        grid=(NPAD // RB, 2),
        in_specs=[
            pl.BlockSpec((2, RB, 64), lambda r, h: (0, r, 0)),
            pl.BlockSpec((2, RB, 64), lambda r, h: (0, r, 0)),
            pl.BlockSpec((RB, 1), lambda r, h: (r, 0)),
            pl.BlockSpec((2, 1, 64), lambda r, h: (0, 0, 0)),
            pl.BlockSpec((2, 1, 64, 32), lambda r, h: (0, h, 0, 0)),
        ],
        out_specs=pl.BlockSpec((1, RB, 32), lambda r, h: (h, r, 0)),
        out_shape=jax.ShapeDtypeStruct((2, NPAD, 32), jnp.float32),
    )(agg1, ys1, dis, b1r, w2q)


def _tc3_body(agg2_ref, ys2_ref, dis_ref, b2_ref, out_ref):
    dis = dis_ref[...]
    o0 = dis * (agg2_ref[0] + ys2_ref[0]) + b2_ref[0]
    o1 = dis * (agg2_ref[1] + ys2_ref[1]) + b2_ref[1]
    out_ref[...] = jnp.concatenate([o0, o1], axis=1)


def _tc3_call(agg2, ys2, dis, b2r):
    return pl.pallas_call(
        _tc3_body,
        grid=(NPAD // RB,),
        in_specs=[
            pl.BlockSpec((2, RB, 32), lambda r: (0, r, 0)),
            pl.BlockSpec((2, RB, 32), lambda r: (0, r, 0)),
            pl.BlockSpec((RB, 1), lambda r: (r, 0)),
            pl.BlockSpec((2, 1, 32), lambda r: (0, 0, 0)),
        ],
        out_specs=pl.BlockSpec((RB, N_CLS), lambda r: (r, 0)),
        out_shape=jax.ShapeDtypeStruct((NPAD, N_CLS), jnp.float32),
    )(agg2, ys2, dis, b2r)


def kernel(x, edge_index, W1, b1, W2, b2):
    with jax.enable_x64(False):
        return _kernel_32(x, edge_index, W1, b1, W2, b2)


def _kernel_32(x, edge_index, W1, b1, W2, b2):
    src = edge_index[0].astype(jnp.int32)
    dst = edge_index[1].astype(jnp.int32)
    x = x.astype(jnp.float32)
    W1 = W1.astype(jnp.float32)
    W2 = W2.astype(jnp.float32)
    b1 = b1.astype(jnp.float32)
    b2 = b2.astype(jnp.float32)
    x_p = jnp.zeros((NPAD, D_IN), jnp.float32).at[:N_NODES].set(x)
    w1h = jnp.stack([W1[:, :64], W1[:, 64:]])
    w2r = W2.reshape(2, 64, N_CLS)
    w2q = jnp.stack([w2r[:, :, :32], w2r[:, :, 32:]], axis=1)
    b1r = b1.reshape(2, 1, 64)
    b2r = b2.reshape(2, 1, 32)

    src3 = src.reshape(NT, AGG_CPT, CH)
    dst3 = dst.reshape(NT, AGG_CPT, CH)
    dstd = dst.reshape(NC * NT, DEG_CPT, CH)

    degp = _deg_call()(dstd).reshape(2, NPAD, 1)
    ys1, dis = _tc1_call(degp, x_p, w1h)
    agg1 = _make_agg(64)(src3, dst3, ys1.reshape(NC * NPAD, 64))
    ys2 = _tc2_call(agg1.reshape(2, NPAD, 64), ys1, dis, b1r, w2q)
    agg2 = _make_agg(32)(src3, dst3, ys2.reshape(NC * NPAD, 32))
    out = _tc3_call(agg2.reshape(2, NPAD, 32), ys2, dis, b2r)
    return out[:N_NODES]


# speedup vs baseline: 331.2694x; 1.1103x over previous
import functools

import jax
import jax.numpy as jnp
from jax import lax
from jax.experimental import pallas as pl
from jax.experimental.pallas import tpu as pltpu
from jax.experimental.pallas import tpu_sc as plsc

N_NODES = 10000
NPAD = 10240
D_IN = 128
D_HID = 128
N_CLS = 64
E_TOT = 320000
E_PAD = 327680
CH = 64
NT = 16
NC = 2
NW = NC * NT
CPT = E_PAD // NW // CH
RPT = NPAD // NT
AGG_R = 4
AGG_NGRP = CPT // AGG_R


@functools.cache
def _mesh():
    return plsc.VectorSubcoreMesh(
        core_axis_name="c", subcore_axis_name="s",
        num_cores=NC, num_subcores=NT)


def _loop(n, body):
    lax.fori_loop(jnp.int32(0), jnp.int32(n), lambda i, _: (body(i), None)[1],
                  None)


def _fill(vref, n, value):
    def body(i):
        vref[pl.ds(i * jnp.int32(16), 16)] = jnp.full((16,), value,
                                                      jnp.float32)
    _loop(n // 16, body)


def _deg_body(dst_hbm, deg_out, acc, idx_all, ones_v, stage_v, sem):
    c = lax.convert_element_type(lax.axis_index("c"), jnp.int32)
    s = lax.convert_element_type(lax.axis_index("s"), jnp.int32)
    zpt = NPAD // NT
    _fill(stage_v, zpt, 0.0)
    _fill(ones_v, CH, 1.0)
    tbase = s * jnp.int32(zpt)
    pltpu.sync_copy(stage_v, acc.at[pl.ds(tbase, zpt)])
    pltpu.sync_copy(dst_hbm.at[c * jnp.int32(NT) + s], idx_all)
    plsc.subcore_barrier()

    def fire(j):
        pltpu.async_copy(ones_v, acc.at[idx_all.at[j]], sem, add=True)

    _loop(CPT, fire)

    def drain(j):
        pltpu.make_async_copy(ones_v, acc.at[idx_all.at[j]], sem).wait()

    _loop(CPT, drain)

    plsc.subcore_barrier()
    pltpu.sync_copy(acc.at[pl.ds(tbase, zpt)], stage_v)
    pltpu.sync_copy(
        stage_v, deg_out.at[pl.ds(c * jnp.int32(NPAD) + tbase, zpt)])


@functools.cache
def _deg_call():
    return pl.kernel(
        _deg_body,
        out_type=jax.ShapeDtypeStruct((NC * NPAD,), jnp.float32),
        mesh=_mesh(),
        scratch_types=[
            pltpu.VMEM_SHARED((NPAD,), jnp.float32),
            pltpu.VMEM((CPT, CH), jnp.int32),
            pltpu.VMEM((CH,), jnp.float32),
            pltpu.VMEM((NPAD // NT,), jnp.float32),
            pltpu.SemaphoreType.DMA,
        ],
    )


def _agg_body(src_hbm, dst_hbm, ys_hbm, out_hbm,
              acc, sidx_all, didx_ring, rows, gsems, ssems, isems):
    c = lax.convert_element_type(lax.axis_index("c"), jnp.int32)
    s = lax.convert_element_type(lax.axis_index("s"), jnp.int32)
    w = c * jnp.int32(NT) + s
    def zrow(r):
        for k in range(D_HID // 16):
            rows[0, r, pl.ds(k * 16, 16)] = jnp.zeros((16,), jnp.float32)

    _loop(CH, zrow)
    tbase = s * jnp.int32(RPT)
    for t in range(RPT // CH):
        pltpu.sync_copy(rows.at[0],
                        acc.at[pl.ds(tbase + jnp.int32(t * CH), CH)])
    pltpu.sync_copy(src_hbm.at[w], sidx_all)
    dbase = w * jnp.int32(CPT)
    plsc.subcore_barrier()

    def sidx(j):
        return sidx_all.at[pl.ds(j * jnp.int32(CH), CH)]

    def gather(j, b):
        pltpu.async_copy(ys_hbm.at[sidx(j)], rows.at[b], gsems.at[b])

    def gather_wait(j, b):
        pltpu.make_async_copy(
            ys_hbm.at[sidx(j)], rows.at[b], gsems.at[b]).wait()

    def didx_load(j, b):
        pltpu.async_copy(dst_hbm.at[dbase + j], didx_ring.at[b],
                         isems.at[b])

    def didx_wait(j, b):
        pltpu.make_async_copy(dst_hbm.at[dbase + j], didx_ring.at[b],
                              isems.at[b]).wait()

    def scatter(j, b):
        pltpu.async_copy(rows.at[b], acc.at[didx_ring.at[b]], ssems.at[b],
                         add=True)

    def scatter_wait(j, b):
        pltpu.make_async_copy(
            rows.at[b], acc.at[didx_ring.at[b]], ssems.at[b]).wait()

    for b in range(AGG_R):
        didx_load(jnp.int32(b), b)
        gather(jnp.int32(b), b)

    def grp(t):
        jb = t * jnp.int32(AGG_R)
        for b in range(AGG_R):
            gather_wait(jb + jnp.int32(b), b)
            didx_wait(jb + jnp.int32(b), b)
            scatter(jb + jnp.int32(b), b)
        for b in range(AGG_R):
            scatter_wait(jb + jnp.int32(b), b)
            didx_load(jb + jnp.int32(AGG_R + b), b)
            gather(jb + jnp.int32(AGG_R + b), b)

    _loop(AGG_NGRP - 1, grp)

    jb = jnp.int32((AGG_NGRP - 1) * AGG_R)
    for b in range(AGG_R):
        gather_wait(jb + jnp.int32(b), b)
        didx_wait(jb + jnp.int32(b), b)
        scatter(jb + jnp.int32(b), b)
    for b in range(AGG_R):
        scatter_wait(jb + jnp.int32(b), b)

    plsc.subcore_barrier()
    obase = c * jnp.int32(NPAD) + tbase
    for t in range(RPT // CH):
        pltpu.sync_copy(acc.at[pl.ds(tbase + jnp.int32(t * CH), CH)],
                        rows.at[0])
        pltpu.sync_copy(rows.at[0],
                        out_hbm.at[pl.ds(obase + jnp.int32(t * CH), CH)])


@functools.cache
def _agg_call():
    return pl.kernel(
        _agg_body,
        out_type=jax.ShapeDtypeStruct((NC * NPAD, D_HID), jnp.float32),
        mesh=_mesh(),
        scratch_types=[
            pltpu.VMEM_SHARED((NPAD, D_HID), jnp.float32),
            pltpu.VMEM((CPT * CH,), jnp.int32),
            pltpu.VMEM((AGG_R, CH), jnp.int32),
            pltpu.VMEM((AGG_R, CH, D_HID), jnp.float32),
            pltpu.SemaphoreType.DMA((AGG_R,)),
            pltpu.SemaphoreType.DMA((AGG_R,)),
            pltpu.SemaphoreType.DMA((AGG_R,)),
        ],
    )


def _tc1a_body(x_ref, w_ref, xw_ref):
    xw_ref[pl.ds(0, N_NODES), :] = jnp.dot(
        x_ref[...], w_ref[...], preferred_element_type=jnp.float32)


def _tc1a_call(x, w1):
    return pl.pallas_call(
        _tc1a_body,
        out_shape=jax.ShapeDtypeStruct((NPAD, D_HID), jnp.float32),
    )(x, w1)


def _tc1b_body(degp_ref, xw_ref, ys_ref, dis_ref):
    deg = degp_ref[0] + degp_ref[1] + 1.0
    dis = lax.rsqrt(deg)
    dis_ref[...] = dis
    ys_ref[...] = xw_ref[...] * dis


def _tc1b_call(degp, xw):
    return pl.pallas_call(
        _tc1b_body,
        out_shape=[
            jax.ShapeDtypeStruct((NPAD, D_HID), jnp.float32),
            jax.ShapeDtypeStruct((NPAD, 1), jnp.float32),
        ],
    )(degp, xw)


def _tc2_body(agg_ref, ys1_ref, dis_ref, b1_ref, zs_ref):
    dis = dis_ref[...]
    h = jnp.maximum(dis * (agg_ref[0] + agg_ref[1] + ys1_ref[...])
                    + b1_ref[...], 0.0)
    zs_ref[...] = h * dis


def _tc2_call(agg1, ys1, dis, b1r):
    return pl.pallas_call(
        _tc2_body,
        out_shape=jax.ShapeDtypeStruct((NPAD, D_HID), jnp.float32),
    )(agg1, ys1, dis, b1r)


def _tc3_body(agg_ref, zs_ref, dis_ref, w2_ref, b2_ref, out_ref):
    u = dis_ref[...] * (agg_ref[0] + agg_ref[1] + zs_ref[...])
    out_ref[...] = jnp.dot(u, w2_ref[...],
                           preferred_element_type=jnp.float32) + b2_ref[...]


def _tc3_call(agg2, zs, dis, w2, b2r):
    return pl.pallas_call(
        _tc3_body,
        out_shape=jax.ShapeDtypeStruct((NPAD, N_CLS), jnp.float32),
    )(agg2, zs, dis, w2, b2r)


def kernel(x, edge_index, W1, b1, W2, b2):
    with jax.enable_x64(False):
        return _kernel_32(x, edge_index, W1, b1, W2, b2)


def _kernel_32(x, edge_index, W1, b1, W2, b2):
    src = edge_index[0].astype(jnp.int32)
    dst = edge_index[1].astype(jnp.int32)
    x = x.astype(jnp.float32)
    W1 = W1.astype(jnp.float32)
    W2 = W2.astype(jnp.float32)
    b1r = b1.astype(jnp.float32).reshape(1, D_HID)
    b2r = b2.astype(jnp.float32).reshape(1, N_CLS)

    pad_idx = N_NODES + jnp.arange(E_PAD - E_TOT, dtype=jnp.int32) % (
        NPAD - N_NODES)
    srcp = jnp.concatenate([src, pad_idx])
    dstp = jnp.concatenate([dst, pad_idx])
    src2 = srcp.reshape(NW, CPT * CH)
    dst2 = dstp.reshape(NW * CPT, CH)
    dst3 = dstp.reshape(NW, CPT, CH)

    degp = _deg_call()(dst3).reshape(2, NPAD, 1)
    xw1 = _tc1a_call(x, W1)
    ys1, dis = _tc1b_call(degp, xw1)
    agg1 = _agg_call()(src2, dst2, ys1).reshape(2, NPAD, D_HID)
    zs = _tc2_call(agg1, ys1, dis, b1r)
    agg2 = _agg_call()(src2, dst2, zs).reshape(2, NPAD, D_HID)
    out = _tc3_call(agg2, zs, dis, W2, b2r)
    return out[:N_NODES]


# speedup vs baseline: 338.8772x; 1.0230x over previous
import functools

import jax
import jax.numpy as jnp
from jax import lax
from jax.experimental import pallas as pl
from jax.experimental.pallas import tpu as pltpu
from jax.experimental.pallas import tpu_sc as plsc

N_NODES = 10000
NPAD = 10240
D_IN = 128
D_HID = 128
N_CLS = 64
E_TOT = 320000
E_PAD = 327680
CH = 64
NT = 16
NC = 2
NW = NC * NT
CPT = E_PAD // NW // CH
RPT = NPAD // NT
AGG_R = 4
AGG_NGRP = CPT // AGG_R


@functools.cache
def _mesh():
    return plsc.VectorSubcoreMesh(
        core_axis_name="c", subcore_axis_name="s",
        num_cores=NC, num_subcores=NT)


def _loop(n, body):
    lax.fori_loop(jnp.int32(0), jnp.int32(n), lambda i, _: (body(i), None)[1],
                  None)


def _fill(vref, n, value):
    def body(i):
        vref[pl.ds(i * jnp.int32(16), 16)] = jnp.full((16,), value,
                                                      jnp.float32)
    _loop(n // 16, body)


DCPT = E_PAD // NT // CH
NREP = NPAD // NW


def _rsqrt16(x):
    i = plsc.bitcast(x, jnp.int32)
    y = plsc.bitcast(jnp.int32(0x5F3759DF) - (i >> jnp.int32(1)),
                     jnp.float32)
    for _ in range(3):
        y = y * (1.5 - 0.5 * x * y * y)
    return y


def _dis_body(dst_hbm, dis_out, acc, idx_all, ones_v, stage_v, rep_v, sem):
    c = lax.convert_element_type(lax.axis_index("c"), jnp.int32)
    s = lax.convert_element_type(lax.axis_index("s"), jnp.int32)
    w = c * jnp.int32(NT) + s
    zpt = NPAD // NT
    _fill(stage_v, zpt, 0.0)
    _fill(ones_v, CH, 1.0)
    pltpu.sync_copy(stage_v, acc.at[pl.ds(s * jnp.int32(zpt), zpt)])
    pltpu.sync_copy(dst_hbm.at[pl.ds(s * jnp.int32(DCPT), DCPT)], idx_all)
    plsc.subcore_barrier()

    def fire(j):
        pltpu.async_copy(ones_v, acc.at[idx_all.at[j]], sem, add=True)

    _loop(DCPT, fire)

    def drain(j):
        pltpu.make_async_copy(ones_v, acc.at[idx_all.at[j]], sem).wait()

    _loop(DCPT, drain)

    plsc.subcore_barrier()
    pltpu.sync_copy(acc.at[pl.ds(w * jnp.int32(NREP), NREP)],
                    stage_v.at[pl.ds(0, NREP)])

    def rsq(i):
        sl = pl.ds(i * jnp.int32(16), 16)
        stage_v[sl] = _rsqrt16(stage_v[sl] + 1.0)

    _loop(NREP // 16, rsq)

    for g in range(NREP // CH):
        for m in range(CH // 16):
            v = stage_v[pl.ds((g * CH // 16 + m) * 16, 16)]
            for l in range(16):
                val = jnp.full((16,), v[l], jnp.float32)
                for t in range(8):
                    rep_v[m * 16 + l, pl.ds(t * 16, 16)] = val
        pltpu.sync_copy(
            rep_v, dis_out.at[pl.ds(w * jnp.int32(NREP) + jnp.int32(g * CH),
                                    CH)])


@functools.cache
def _dis_call():
    return pl.kernel(
        _dis_body,
        out_type=jax.ShapeDtypeStruct((NPAD, D_HID), jnp.float32),
        mesh=_mesh(),
        scratch_types=[
            pltpu.VMEM_SHARED((NPAD,), jnp.float32),
            pltpu.VMEM((DCPT, CH), jnp.int32),
            pltpu.VMEM((CH,), jnp.float32),
            pltpu.VMEM((NPAD // NT,), jnp.float32),
            pltpu.VMEM((CH, D_HID), jnp.float32),
            pltpu.SemaphoreType.DMA,
        ],
        compiler_params=pltpu.CompilerParams(needs_layout_passes=False),
    )


def _agg_body(src_hbm, dst_hbm, ys_hbm, out_hbm,
              acc, sidx_all, didx_ring, rows, gsems, ssems, isems):
    c = lax.convert_element_type(lax.axis_index("c"), jnp.int32)
    s = lax.convert_element_type(lax.axis_index("s"), jnp.int32)
    w = c * jnp.int32(NT) + s
    def zrow(r):
        for k in range(D_HID // 16):
            rows[0, r, pl.ds(k * 16, 16)] = jnp.zeros((16,), jnp.float32)

    _loop(CH, zrow)
    tbase = s * jnp.int32(RPT)
    for t in range(RPT // CH):
        pltpu.sync_copy(rows.at[0],
                        acc.at[pl.ds(tbase + jnp.int32(t * CH), CH)])
    pltpu.sync_copy(src_hbm.at[w], sidx_all)
    dbase = w * jnp.int32(CPT)
    plsc.subcore_barrier()

    def sidx(j):
        return sidx_all.at[pl.ds(j * jnp.int32(CH), CH)]

    def gather(j, b):
        pltpu.async_copy(ys_hbm.at[sidx(j)], rows.at[b], gsems.at[b])

    def gather_wait(j, b):
        pltpu.make_async_copy(
            ys_hbm.at[sidx(j)], rows.at[b], gsems.at[b]).wait()

    def didx_load(j, b):
        pltpu.async_copy(dst_hbm.at[dbase + j], didx_ring.at[b],
                         isems.at[b])

    def didx_wait(j, b):
        pltpu.make_async_copy(dst_hbm.at[dbase + j], didx_ring.at[b],
                              isems.at[b]).wait()

    def scatter(j, b):
        pltpu.async_copy(rows.at[b], acc.at[didx_ring.at[b]], ssems.at[b],
                         add=True)

    def scatter_wait(j, b):
        pltpu.make_async_copy(
            rows.at[b], acc.at[didx_ring.at[b]], ssems.at[b]).wait()

    for b in range(AGG_R):
        didx_load(jnp.int32(b), b)
        gather(jnp.int32(b), b)

    def grp(t):
        jb = t * jnp.int32(AGG_R)
        for b in range(AGG_R):
            gather_wait(jb + jnp.int32(b), b)
            didx_wait(jb + jnp.int32(b), b)
            scatter(jb + jnp.int32(b), b)
        for b in range(AGG_R):
            scatter_wait(jb + jnp.int32(b), b)
            didx_load(jb + jnp.int32(AGG_R + b), b)
            gather(jb + jnp.int32(AGG_R + b), b)

    _loop(AGG_NGRP - 1, grp)

    jb = jnp.int32((AGG_NGRP - 1) * AGG_R)
    for b in range(AGG_R):
        gather_wait(jb + jnp.int32(b), b)
        didx_wait(jb + jnp.int32(b), b)
        scatter(jb + jnp.int32(b), b)
    for b in range(AGG_R):
        scatter_wait(jb + jnp.int32(b), b)

    plsc.subcore_barrier()
    obase = c * jnp.int32(NPAD) + tbase
    for t in range(RPT // CH):
        pltpu.sync_copy(acc.at[pl.ds(tbase + jnp.int32(t * CH), CH)],
                        rows.at[0])
        pltpu.sync_copy(rows.at[0],
                        out_hbm.at[pl.ds(obase + jnp.int32(t * CH), CH)])


@functools.cache
def _agg_call():
    return pl.kernel(
        _agg_body,
        out_type=jax.ShapeDtypeStruct((NC * NPAD, D_HID), jnp.float32),
        mesh=_mesh(),
        scratch_types=[
            pltpu.VMEM_SHARED((NPAD, D_HID), jnp.float32),
            pltpu.VMEM((CPT * CH,), jnp.int32),
            pltpu.VMEM((AGG_R, CH), jnp.int32),
            pltpu.VMEM((AGG_R, CH, D_HID), jnp.float32),
            pltpu.SemaphoreType.DMA((AGG_R,)),
            pltpu.SemaphoreType.DMA((AGG_R,)),
            pltpu.SemaphoreType.DMA((AGG_R,)),
        ],
    )


def _tc1a_body(x_ref, w_ref, xw_ref):
    xw_ref[pl.ds(0, N_NODES), :] = jnp.dot(
        x_ref[...], w_ref[...], preferred_element_type=jnp.float32)


def _tc1a_call(x, w1):
    return pl.pallas_call(
        _tc1a_body,
        out_shape=jax.ShapeDtypeStruct((NPAD, D_HID), jnp.float32),
    )(x, w1)


def _tc1b_body(dis_ref, xw_ref, ys_ref):
    ys_ref[...] = xw_ref[...] * dis_ref[...]


def _tc1b_call(dis_rep, xw):
    return pl.pallas_call(
        _tc1b_body,
        out_shape=jax.ShapeDtypeStruct((NPAD, D_HID), jnp.float32),
    )(dis_rep, xw)


def _tc2_body(agg_ref, ys1_ref, dis_ref, b1_ref, zs_ref):
    dis = dis_ref[...]
    h = jnp.maximum(dis * (agg_ref[0] + agg_ref[1] + ys1_ref[...])
                    + b1_ref[...], 0.0)
    zs_ref[...] = h * dis


def _tc2_call(agg1, ys1, dis_rep, b1r):
    return pl.pallas_call(
        _tc2_body,
        out_shape=jax.ShapeDtypeStruct((NPAD, D_HID), jnp.float32),
    )(agg1, ys1, dis_rep, b1r)


def _tc3_body(agg_ref, zs_ref, dis_ref, w2_ref, b2_ref, out_ref):
    u = dis_ref[...] * (agg_ref[0] + agg_ref[1] + zs_ref[...])
    out_ref[...] = jnp.dot(u[:N_NODES, :], w2_ref[...],
                           preferred_element_type=jnp.float32) + b2_ref[...]


def _tc3_call(agg2, zs, dis_rep, w2, b2r):
    return pl.pallas_call(
        _tc3_body,
        out_shape=jax.ShapeDtypeStruct((N_NODES, N_CLS), jnp.float32),
    )(agg2, zs, dis_rep, w2, b2r)


def kernel(x, edge_index, W1, b1, W2, b2):
    with jax.enable_x64(False):
        return _kernel_32(x, edge_index, W1, b1, W2, b2)


def _kernel_32(x, edge_index, W1, b1, W2, b2):
    src = edge_index[0].astype(jnp.int32)
    dst = edge_index[1].astype(jnp.int32)
    x = x.astype(jnp.float32)
    W1 = W1.astype(jnp.float32)
    W2 = W2.astype(jnp.float32)
    b1r = b1.astype(jnp.float32).reshape(1, D_HID)
    b2r = b2.astype(jnp.float32).reshape(1, N_CLS)

    pad_idx = N_NODES + jnp.arange(E_PAD - E_TOT, dtype=jnp.int32) % (
        NPAD - N_NODES)
    srcp = jnp.concatenate([src, pad_idx])
    dstp = jnp.concatenate([dst, pad_idx])
    src2 = srcp.reshape(NW, CPT * CH)
    dst2 = dstp.reshape(NW * CPT, CH)

    dis_rep = _dis_call()(dst2)
    xw1 = _tc1a_call(x, W1)
    ys1 = _tc1b_call(dis_rep, xw1)
    agg1 = _agg_call()(src2, dst2, ys1).reshape(2, NPAD, D_HID)
    zs = _tc2_call(agg1, ys1, dis_rep, b1r)
    agg2 = _agg_call()(src2, dst2, zs).reshape(2, NPAD, D_HID)
    return _tc3_call(agg2, zs, dis_rep, W2, b2r)


# speedup vs baseline: 391.1436x; 1.1542x over previous
import functools

import jax
import jax.numpy as jnp
from jax import lax
from jax.experimental import pallas as pl
from jax.experimental.pallas import tpu as pltpu
from jax.experimental.pallas import tpu_sc as plsc

N_NODES = 10000
NPAD = 10240
D_IN = 128
D_HID = 128
N_CLS = 64
E_TOT = 320000
E_PAD = 327680
CH = 64
NT = 16
NC = 2
NW = NC * NT
CPT = E_PAD // NW // CH
RPT = NPAD // NT
AGG_R = 4
AGG_NGRP = CPT // AGG_R


@functools.cache
def _mesh():
    return plsc.VectorSubcoreMesh(
        core_axis_name="c", subcore_axis_name="s",
        num_cores=NC, num_subcores=NT)


def _loop(n, body):
    lax.fori_loop(jnp.int32(0), jnp.int32(n), lambda i, _: (body(i), None)[1],
                  None)


def _fill(vref, n, value):
    def body(i):
        vref[pl.ds(i * jnp.int32(16), 16)] = jnp.full((16,), value,
                                                      jnp.float32)
    _loop(n // 16, body)


DCPT = E_PAD // NT // CH
NREP = NPAD // NW


def _rsqrt16(x):
    i = plsc.bitcast(x, jnp.int32)
    y = plsc.bitcast(jnp.int32(0x5F3759DF) - (i >> jnp.int32(1)),
                     jnp.float32)
    for _ in range(3):
        y = y * (1.5 - 0.5 * x * y * y)
    return y


def _dis_body(dst_hbm, dis_out, acc, idx_all, ones_v, stage_v, rep_v, sem):
    c = lax.convert_element_type(lax.axis_index("c"), jnp.int32)
    s = lax.convert_element_type(lax.axis_index("s"), jnp.int32)
    w = c * jnp.int32(NT) + s
    zpt = NPAD // NT
    _fill(stage_v, zpt, 0.0)
    _fill(ones_v, CH, 1.0)
    pltpu.sync_copy(stage_v, acc.at[pl.ds(s * jnp.int32(zpt), zpt)])
    pltpu.sync_copy(dst_hbm.at[pl.ds(s * jnp.int32(DCPT), DCPT)], idx_all)
    plsc.subcore_barrier()

    def fire(j):
        pltpu.async_copy(ones_v, acc.at[idx_all.at[j]], sem, add=True)

    _loop(DCPT, fire)

    def drain(j):
        pltpu.make_async_copy(ones_v, acc.at[idx_all.at[j]], sem).wait()

    _loop(DCPT, drain)

    plsc.subcore_barrier()
    pltpu.sync_copy(acc.at[pl.ds(w * jnp.int32(NREP), NREP)],
                    stage_v.at[pl.ds(0, NREP)])

    def rsq(i):
        sl = pl.ds(i * jnp.int32(16), 16)
        stage_v[sl] = _rsqrt16(stage_v[sl] + 1.0)

    _loop(NREP // 16, rsq)

    for g in range(NREP // CH):
        for m in range(CH // 16):
            v = stage_v[pl.ds((g * CH // 16 + m) * 16, 16)]
            for l in range(16):
                val = jnp.full((16,), v[l], jnp.float32)
                for t in range(8):
                    rep_v[m * 16 + l, pl.ds(t * 16, 16)] = val
        pltpu.sync_copy(
            rep_v, dis_out.at[pl.ds(w * jnp.int32(NREP) + jnp.int32(g * CH),
                                    CH)])


@functools.cache
def _dis_call():
    return pl.kernel(
        _dis_body,
        out_type=jax.ShapeDtypeStruct((NPAD, D_HID), jnp.float32),
        mesh=_mesh(),
        scratch_types=[
            pltpu.VMEM_SHARED((NPAD,), jnp.float32),
            pltpu.VMEM((DCPT, CH), jnp.int32),
            pltpu.VMEM((CH,), jnp.float32),
            pltpu.VMEM((NPAD // NT,), jnp.float32),
            pltpu.VMEM((CH, D_HID), jnp.float32),
            pltpu.SemaphoreType.DMA,
        ],
        compiler_params=pltpu.CompilerParams(needs_layout_passes=False),
    )


def _agg_body(dh, ch, rr, src_hbm, dst_hbm, ys_hbm, out_hbm,
              acc, sidx_all, didx_ring, rows, gsems, ssems, isems):
    cpt = E_PAD // NW // ch
    ngrp = cpt // rr
    c = lax.convert_element_type(lax.axis_index("c"), jnp.int32)
    s = lax.convert_element_type(lax.axis_index("s"), jnp.int32)
    w = c * jnp.int32(NT) + s
    def zrow(r):
        for k in range(dh // 16):
            rows[0, r, pl.ds(k * 16, 16)] = jnp.zeros((16,), jnp.float32)

    _loop(ch, zrow)
    tbase = s * jnp.int32(RPT)
    for t in range(RPT // ch):
        pltpu.sync_copy(rows.at[0],
                        acc.at[pl.ds(tbase + jnp.int32(t * ch), ch)])
    pltpu.sync_copy(src_hbm.at[w], sidx_all)
    dbase = w * jnp.int32(cpt)
    plsc.subcore_barrier()

    def sidx(j):
        return sidx_all.at[pl.ds(j * jnp.int32(ch), ch)]

    def gather(j, b):
        pltpu.async_copy(ys_hbm.at[sidx(j)], rows.at[b], gsems.at[b])

    def gather_wait(j, b):
        pltpu.make_async_copy(
            ys_hbm.at[sidx(j)], rows.at[b], gsems.at[b]).wait()

    def didx_load(j, b):
        pltpu.async_copy(dst_hbm.at[dbase + j], didx_ring.at[b],
                         isems.at[b])

    def didx_wait(j, b):
        pltpu.make_async_copy(dst_hbm.at[dbase + j], didx_ring.at[b],
                              isems.at[b]).wait()

    def scatter(j, b):
        pltpu.async_copy(rows.at[b], acc.at[didx_ring.at[b]], ssems.at[b],
                         add=True)

    def scatter_wait(j, b):
        pltpu.make_async_copy(
            rows.at[b], acc.at[didx_ring.at[b]], ssems.at[b]).wait()

    for b in range(rr):
        didx_load(jnp.int32(b), b)
        gather(jnp.int32(b), b)

    def grp(t):
        jb = t * jnp.int32(rr)
        for b in range(rr):
            gather_wait(jb + jnp.int32(b), b)
            didx_wait(jb + jnp.int32(b), b)
            scatter(jb + jnp.int32(b), b)
        for b in range(rr):
            scatter_wait(jb + jnp.int32(b), b)
            didx_load(jb + jnp.int32(rr + b), b)
            gather(jb + jnp.int32(rr + b), b)

    _loop(ngrp - 1, grp)

    jb = jnp.int32((ngrp - 1) * rr)
    for b in range(rr):
        gather_wait(jb + jnp.int32(b), b)
        didx_wait(jb + jnp.int32(b), b)
        scatter(jb + jnp.int32(b), b)
    for b in range(rr):
        scatter_wait(jb + jnp.int32(b), b)

    plsc.subcore_barrier()
    obase = c * jnp.int32(NPAD) + tbase
    for t in range(RPT // ch):
        pltpu.sync_copy(acc.at[pl.ds(tbase + jnp.int32(t * ch), ch)],
                        rows.at[0])
        pltpu.sync_copy(rows.at[0],
                        out_hbm.at[pl.ds(obase + jnp.int32(t * ch), ch)])


@functools.cache
def _agg_call(dh, ch, rr, tc_tiling):
    cpt = E_PAD // NW // ch
    return pl.kernel(
        functools.partial(_agg_body, dh, ch, rr),
        out_type=jax.ShapeDtypeStruct((NC * NPAD, dh), jnp.float32),
        mesh=_mesh(),
        scratch_types=[
            pltpu.VMEM_SHARED((NPAD, dh), jnp.float32),
            pltpu.VMEM((cpt * ch,), jnp.int32),
            pltpu.VMEM((rr, ch), jnp.int32),
            pltpu.VMEM((rr, ch, dh), jnp.float32),
            pltpu.SemaphoreType.DMA((rr,)),
            pltpu.SemaphoreType.DMA((rr,)),
            pltpu.SemaphoreType.DMA((rr,)),
        ],
        compiler_params=pltpu.CompilerParams(use_tc_tiling_on_sc=tc_tiling),
    )


def _tc1a_body(x_ref, w_ref, xw_ref):
    xw_ref[pl.ds(0, N_NODES), :] = jnp.dot(
        x_ref[...], w_ref[...], preferred_element_type=jnp.float32)


def _tc1a_call(x, w1):
    return pl.pallas_call(
        _tc1a_body,
        out_shape=jax.ShapeDtypeStruct((NPAD, D_HID), jnp.float32),
    )(x, w1)


def _tc1b_body(dis_ref, xw_ref, ys_ref):
    ys_ref[...] = xw_ref[...] * dis_ref[...]


def _tc1b_call(dis_rep, xw):
    return pl.pallas_call(
        _tc1b_body,
        out_shape=jax.ShapeDtypeStruct((NPAD, D_HID), jnp.float32),
    )(dis_rep, xw)


def _tc2_body(agg_ref, ys1_ref, dis_ref, b1_ref, w2_ref, ys2_ref):
    dis = dis_ref[...]
    h = jnp.maximum(dis * (agg_ref[0] + agg_ref[1] + ys1_ref[...])
                    + b1_ref[...], 0.0)
    ys2_ref[...] = jnp.dot(h, w2_ref[...],
                           preferred_element_type=jnp.float32) * dis[:, :N_CLS]


def _tc2_call(agg1, ys1, dis_rep, b1r, w2):
    return pl.pallas_call(
        _tc2_body,
        out_shape=jax.ShapeDtypeStruct((NPAD, N_CLS), jnp.float32),
    )(agg1, ys1, dis_rep, b1r, w2)


def _tc3_body(agg_ref, ys2_ref, dis_ref, b2_ref, out_ref):
    u = dis_ref[...][:N_NODES, :N_CLS] * (
        agg_ref[0][:N_NODES] + agg_ref[1][:N_NODES] + ys2_ref[...][:N_NODES])
    out_ref[...] = u + b2_ref[...]


def _tc3_call(agg2, ys2, dis_rep, b2r):
    return pl.pallas_call(
        _tc3_body,
        out_shape=jax.ShapeDtypeStruct((N_NODES, N_CLS), jnp.float32),
    )(agg2, ys2, dis_rep, b2r)


def kernel(x, edge_index, W1, b1, W2, b2):
    with jax.enable_x64(False):
        return _kernel_32(x, edge_index, W1, b1, W2, b2)


def _kernel_32(x, edge_index, W1, b1, W2, b2):
    src = edge_index[0].astype(jnp.int32)
    dst = edge_index[1].astype(jnp.int32)
    x = x.astype(jnp.float32)
    W1 = W1.astype(jnp.float32)
    W2 = W2.astype(jnp.float32)
    b1r = b1.astype(jnp.float32).reshape(1, D_HID)
    b2r = b2.astype(jnp.float32).reshape(1, N_CLS)

    pad_idx = N_NODES + jnp.arange(E_PAD - E_TOT, dtype=jnp.int32) % (
        NPAD - N_NODES)
    srcp = jnp.concatenate([src, pad_idx])
    dstp = jnp.concatenate([dst, pad_idx])
    src2 = srcp.reshape(NW, CPT * CH)
    dst2 = dstp.reshape(NW * CPT, CH)

    dst2b = dstp.reshape(NW * (E_PAD // NW // 128), 128)

    dis_rep = _dis_call()(dst2)
    xw1 = _tc1a_call(x, W1)
    ys1 = _tc1b_call(dis_rep, xw1)
    agg1 = _agg_call(D_HID, CH, AGG_R, True)(src2, dst2, ys1)
    ys2 = _tc2_call(agg1.reshape(2, NPAD, D_HID), ys1, dis_rep, b1r, W2)
    agg2 = _agg_call(N_CLS, 128, 5, False)(src2, dst2b, ys2)
    return _tc3_call(agg2.reshape(2, NPAD, N_CLS), ys2, dis_rep, b2r)


# speedup vs baseline: 395.3483x; 1.0107x over previous
import functools

import jax
import jax.numpy as jnp
from jax import lax
from jax.experimental import pallas as pl
from jax.experimental.pallas import tpu as pltpu
from jax.experimental.pallas import tpu_sc as plsc

N_NODES = 10000
NPAD = 10240
D_IN = 128
D_HID = 128
N_CLS = 64
E_TOT = 320000
E_PAD = 327680
CH = 64
NT = 16
NC = 2
NW = NC * NT
CPT = E_PAD // NW // CH
RPT = NPAD // NT
AGG_R = 4
AGG_NGRP = CPT // AGG_R


@functools.cache
def _mesh():
    return plsc.VectorSubcoreMesh(
        core_axis_name="c", subcore_axis_name="s",
        num_cores=NC, num_subcores=NT)


def _loop(n, body):
    lax.fori_loop(jnp.int32(0), jnp.int32(n), lambda i, _: (body(i), None)[1],
                  None)


def _fill(vref, n, value):
    def body(i):
        vref[pl.ds(i * jnp.int32(16), 16)] = jnp.full((16,), value,
                                                      jnp.float32)
    _loop(n // 16, body)


CHD = 128
DCPT = E_PAD // NT // CHD
NREP = NPAD // NW


def _rsqrt16(x):
    i = plsc.bitcast(x, jnp.int32)
    y = plsc.bitcast(jnp.int32(0x5F3759DF) - (i >> jnp.int32(1)),
                     jnp.float32)
    for _ in range(3):
        y = y * (1.5 - 0.5 * x * y * y)
    return y


def _dis_body(dst_hbm, dis_out, acc, idx_all, ones_v, stage_v, rep_v, sem):
    c = lax.convert_element_type(lax.axis_index("c"), jnp.int32)
    s = lax.convert_element_type(lax.axis_index("s"), jnp.int32)
    w = c * jnp.int32(NT) + s
    zpt = NPAD // NT
    _fill(stage_v, zpt, 0.0)
    _fill(ones_v, CHD, 1.0)
    pltpu.sync_copy(stage_v, acc.at[pl.ds(s * jnp.int32(zpt), zpt)])
    pltpu.sync_copy(dst_hbm.at[pl.ds(s * jnp.int32(DCPT), DCPT)], idx_all)
    plsc.subcore_barrier()

    def fire(j):
        pltpu.async_copy(ones_v, acc.at[idx_all.at[j]], sem, add=True)

    _loop(DCPT, fire)

    def drain(j):
        pltpu.make_async_copy(ones_v, acc.at[idx_all.at[j]], sem).wait()

    _loop(DCPT, drain)

    plsc.subcore_barrier()
    pltpu.sync_copy(acc.at[pl.ds(w * jnp.int32(NREP), NREP)],
                    stage_v.at[pl.ds(0, NREP)])

    def rsq(i):
        sl = pl.ds(i * jnp.int32(16), 16)
        stage_v[sl] = _rsqrt16(stage_v[sl] + 1.0)

    _loop(NREP // 16, rsq)

    for g in range(NREP // CH):
        for m in range(CH // 16):
            v = stage_v[pl.ds((g * CH // 16 + m) * 16, 16)]
            for l in range(16):
                val = jnp.full((16,), v[l], jnp.float32)
                for t in range(8):
                    rep_v[m * 16 + l, pl.ds(t * 16, 16)] = val
        pltpu.sync_copy(
            rep_v, dis_out.at[pl.ds(w * jnp.int32(NREP) + jnp.int32(g * CH),
                                    CH)])


@functools.cache
def _dis_call():
    return pl.kernel(
        _dis_body,
        out_type=jax.ShapeDtypeStruct((NPAD, D_HID), jnp.float32),
        mesh=_mesh(),
        scratch_types=[
            pltpu.VMEM_SHARED((NPAD,), jnp.float32),
            pltpu.VMEM((DCPT, CHD), jnp.int32),
            pltpu.VMEM((CHD,), jnp.float32),
            pltpu.VMEM((NPAD // NT,), jnp.float32),
            pltpu.VMEM((CH, D_HID), jnp.float32),
            pltpu.SemaphoreType.DMA,
        ],
        compiler_params=pltpu.CompilerParams(needs_layout_passes=False),
    )


def _agg_body(dh, ch, rr, src_hbm, dst_hbm, ys_hbm, out_hbm,
              acc, sidx_all, didx_ring, rows, gsems, ssems, isems):
    cpt = E_PAD // NW // ch
    ngrp = cpt // rr
    c = lax.convert_element_type(lax.axis_index("c"), jnp.int32)
    s = lax.convert_element_type(lax.axis_index("s"), jnp.int32)
    w = c * jnp.int32(NT) + s
    def zrow(r):
        for k in range(dh // 16):
            rows[0, r, pl.ds(k * 16, 16)] = jnp.zeros((16,), jnp.float32)

    _loop(ch, zrow)
    tbase = s * jnp.int32(RPT)
    for t in range(RPT // ch):
        pltpu.sync_copy(rows.at[0],
                        acc.at[pl.ds(tbase + jnp.int32(t * ch), ch)])
    pltpu.sync_copy(src_hbm.at[w], sidx_all)
    dbase = w * jnp.int32(cpt)
    plsc.subcore_barrier()

    def sidx(j):
        return sidx_all.at[pl.ds(j * jnp.int32(ch), ch)]

    def gather(j, b):
        pltpu.async_copy(ys_hbm.at[sidx(j)], rows.at[b], gsems.at[b])

    def gather_wait(j, b):
        pltpu.make_async_copy(
            ys_hbm.at[sidx(j)], rows.at[b], gsems.at[b]).wait()

    def didx_load(j, b):
        pltpu.async_copy(dst_hbm.at[dbase + j], didx_ring.at[b],
                         isems.at[b])

    def didx_wait(j, b):
        pltpu.make_async_copy(dst_hbm.at[dbase + j], didx_ring.at[b],
                              isems.at[b]).wait()

    def scatter(j, b):
        pltpu.async_copy(rows.at[b], acc.at[didx_ring.at[b]], ssems.at[b],
                         add=True)

    def scatter_wait(j, b):
        pltpu.make_async_copy(
            rows.at[b], acc.at[didx_ring.at[b]], ssems.at[b]).wait()

    for b in range(rr):
        didx_load(jnp.int32(b), b)
        gather(jnp.int32(b), b)

    def grp(t):
        jb = t * jnp.int32(rr)
        for b in range(rr):
            gather_wait(jb + jnp.int32(b), b)
            didx_wait(jb + jnp.int32(b), b)
            scatter(jb + jnp.int32(b), b)
        for b in range(rr):
            scatter_wait(jb + jnp.int32(b), b)
            didx_load(jb + jnp.int32(rr + b), b)
            gather(jb + jnp.int32(rr + b), b)

    _loop(ngrp - 1, grp)

    jb = jnp.int32((ngrp - 1) * rr)
    for b in range(rr):
        gather_wait(jb + jnp.int32(b), b)
        didx_wait(jb + jnp.int32(b), b)
        scatter(jb + jnp.int32(b), b)
    for b in range(rr):
        scatter_wait(jb + jnp.int32(b), b)

    plsc.subcore_barrier()
    obase = c * jnp.int32(NPAD) + tbase
    for t in range(RPT // ch):
        pltpu.sync_copy(acc.at[pl.ds(tbase + jnp.int32(t * ch), ch)],
                        rows.at[0])
        pltpu.sync_copy(rows.at[0],
                        out_hbm.at[pl.ds(obase + jnp.int32(t * ch), ch)])


@functools.cache
def _agg_call(dh, ch, rr, tc_tiling):
    cpt = E_PAD // NW // ch
    return pl.kernel(
        functools.partial(_agg_body, dh, ch, rr),
        out_type=jax.ShapeDtypeStruct((NC * NPAD, dh), jnp.float32),
        mesh=_mesh(),
        scratch_types=[
            pltpu.VMEM_SHARED((NPAD, dh), jnp.float32),
            pltpu.VMEM((cpt * ch,), jnp.int32),
            pltpu.VMEM((rr, ch), jnp.int32),
            pltpu.VMEM((rr, ch, dh), jnp.float32),
            pltpu.SemaphoreType.DMA((rr,)),
            pltpu.SemaphoreType.DMA((rr,)),
            pltpu.SemaphoreType.DMA((rr,)),
        ],
        compiler_params=pltpu.CompilerParams(use_tc_tiling_on_sc=tc_tiling),
    )


def _tc1a_body(x_ref, w_ref, xw_ref):
    xw_ref[pl.ds(0, N_NODES), :] = jnp.dot(
        x_ref[...], w_ref[...], preferred_element_type=jnp.float32)


def _tc1a_call(x, w1):
    return pl.pallas_call(
        _tc1a_body,
        out_shape=jax.ShapeDtypeStruct((NPAD, D_HID), jnp.float32),
    )(x, w1)


def _tc1b_body(dis_ref, xw_ref, ys_ref):
    ys_ref[...] = xw_ref[...] * dis_ref[...]


def _tc1b_call(dis_rep, xw):
    return pl.pallas_call(
        _tc1b_body,
        out_shape=jax.ShapeDtypeStruct((NPAD, D_HID), jnp.float32),
    )(dis_rep, xw)


def _tc2_body(agg_ref, ys1_ref, dis_ref, b1_ref, w2_ref, ys2_ref):
    dis = dis_ref[...]
    h = jnp.maximum(dis * (agg_ref[0] + agg_ref[1] + ys1_ref[...])
                    + b1_ref[...], 0.0)
    ys2_ref[...] = jnp.dot(h, w2_ref[...],
                           preferred_element_type=jnp.float32) * dis[:, :N_CLS]


def _tc2_call(agg1, ys1, dis_rep, b1r, w2):
    return pl.pallas_call(
        _tc2_body,
        out_shape=jax.ShapeDtypeStruct((NPAD, N_CLS), jnp.float32),
    )(agg1, ys1, dis_rep, b1r, w2)


def _tc3_body(agg_ref, ys2_ref, dis_ref, b2_ref, out_ref):
    u = dis_ref[...][:N_NODES, :N_CLS] * (
        agg_ref[0][:N_NODES] + agg_ref[1][:N_NODES] + ys2_ref[...][:N_NODES])
    out_ref[...] = u + b2_ref[...]


def _tc3_call(agg2, ys2, dis_rep, b2r):
    return pl.pallas_call(
        _tc3_body,
        out_shape=jax.ShapeDtypeStruct((N_NODES, N_CLS), jnp.float32),
    )(agg2, ys2, dis_rep, b2r)


def kernel(x, edge_index, W1, b1, W2, b2):
    with jax.enable_x64(False):
        return _kernel_32(x, edge_index, W1, b1, W2, b2)


def _kernel_32(x, edge_index, W1, b1, W2, b2):
    src = edge_index[0].astype(jnp.int32)
    dst = edge_index[1].astype(jnp.int32)
    x = x.astype(jnp.float32)
    W1 = W1.astype(jnp.float32)
    W2 = W2.astype(jnp.float32)
    b1r = b1.astype(jnp.float32).reshape(1, D_HID)
    b2r = b2.astype(jnp.float32).reshape(1, N_CLS)

    pad_idx = N_NODES + jnp.arange(E_PAD - E_TOT, dtype=jnp.int32) % (
        NPAD - N_NODES)
    srcp = jnp.concatenate([src, pad_idx])
    dstp = jnp.concatenate([dst, pad_idx])
    src2 = srcp.reshape(NW, CPT * CH)
    dst2 = dstp.reshape(NW * CPT, CH)

    dst2b = dstp.reshape(NW * (E_PAD // NW // 128), 128)

    dis_rep = _dis_call()(dst2b)
    xw1 = _tc1a_call(x, W1)
    ys1 = _tc1b_call(dis_rep, xw1)
    agg1 = _agg_call(D_HID, CH, AGG_R, True)(src2, dst2, ys1)
    ys2 = _tc2_call(agg1.reshape(2, NPAD, D_HID), ys1, dis_rep, b1r, W2)
    agg2 = _agg_call(N_CLS, 128, 5, False)(src2, dst2b, ys2)
    return _tc3_call(agg2.reshape(2, NPAD, N_CLS), ys2, dis_rep, b2r)


# speedup vs baseline: 397.3133x; 1.0050x over previous
import functools

import jax
import jax.numpy as jnp
from jax import lax
from jax.experimental import pallas as pl
from jax.experimental.pallas import tpu as pltpu
from jax.experimental.pallas import tpu_sc as plsc

N_NODES = 10000
NPAD = 10240
D_IN = 128
D_HID = 128
N_CLS = 64
E_TOT = 320000
E_PAD = 327680
CH = 64
NT = 16
NC = 2
NW = NC * NT
CPT = E_PAD // NW // CH
RPT = NPAD // NT
AGG_R = 4
AGG_NGRP = CPT // AGG_R


@functools.cache
def _mesh():
    return plsc.VectorSubcoreMesh(
        core_axis_name="c", subcore_axis_name="s",
        num_cores=NC, num_subcores=NT)


def _loop(n, body):
    lax.fori_loop(jnp.int32(0), jnp.int32(n), lambda i, _: (body(i), None)[1],
                  None)


def _fill(vref, n, value):
    def body(i):
        vref[pl.ds(i * jnp.int32(16), 16)] = jnp.full((16,), value,
                                                      jnp.float32)
    _loop(n // 16, body)


CHD = 128
DCPT = E_PAD // NT // CHD
NREP = NPAD // NW


def _rsqrt16(x):
    i = plsc.bitcast(x, jnp.int32)
    y = plsc.bitcast(jnp.int32(0x5F3759DF) - (i >> jnp.int32(1)),
                     jnp.float32)
    for _ in range(3):
        y = y * (1.5 - 0.5 * x * y * y)
    return y


def _dis_body(dst_hbm, dis_out, acc, idx_all, ones_v, stage_v, rep_v, sem):
    c = lax.convert_element_type(lax.axis_index("c"), jnp.int32)
    s = lax.convert_element_type(lax.axis_index("s"), jnp.int32)
    w = c * jnp.int32(NT) + s
    zpt = NPAD // NT
    _fill(stage_v, zpt, 0.0)
    _fill(ones_v, CHD, 1.0)
    pltpu.sync_copy(stage_v, acc.at[pl.ds(s * jnp.int32(zpt), zpt)])
    pltpu.sync_copy(dst_hbm.at[pl.ds(s * jnp.int32(DCPT), DCPT)], idx_all)
    plsc.subcore_barrier()

    def fire(j):
        pltpu.async_copy(ones_v, acc.at[idx_all.at[j]], sem, add=True)

    _loop(DCPT, fire)

    def drain(j):
        pltpu.make_async_copy(ones_v, acc.at[idx_all.at[j]], sem).wait()

    _loop(DCPT, drain)

    plsc.subcore_barrier()
    pltpu.sync_copy(acc.at[pl.ds(w * jnp.int32(NREP), NREP)],
                    stage_v.at[pl.ds(0, NREP)])

    def rsq(i):
        sl = pl.ds(i * jnp.int32(16), 16)
        stage_v[sl] = _rsqrt16(stage_v[sl] + 1.0)

    _loop(NREP // 16, rsq)

    for g in range(NREP // CH):
        for m in range(CH // 16):
            v = stage_v[pl.ds((g * CH // 16 + m) * 16, 16)]
            for l in range(16):
                val = jnp.full((16,), v[l], jnp.float32)
                for t in range(8):
                    rep_v[m * 16 + l, pl.ds(t * 16, 16)] = val
        pltpu.sync_copy(
            rep_v, dis_out.at[pl.ds(w * jnp.int32(NREP) + jnp.int32(g * CH),
                                    CH)])


@functools.cache
def _dis_call():
    return pl.kernel(
        _dis_body,
        out_type=jax.ShapeDtypeStruct((NPAD, D_HID), jnp.float32),
        mesh=_mesh(),
        scratch_types=[
            pltpu.VMEM_SHARED((NPAD,), jnp.float32),
            pltpu.VMEM((DCPT, CHD), jnp.int32),
            pltpu.VMEM((CHD,), jnp.float32),
            pltpu.VMEM((NPAD // NT,), jnp.float32),
            pltpu.VMEM((CH, D_HID), jnp.float32),
            pltpu.SemaphoreType.DMA,
        ],
        compiler_params=pltpu.CompilerParams(needs_layout_passes=False),
    )


def _agg_body(dh, ch, rr, phases, src_hbm, dst_hbm, ys_hbm, out_hbm,
              acc, sidx_all, didx_ring, rows, gsems, ssems, isems):
    cpt = E_PAD // NW // ch
    pcpt = cpt // phases
    ngrp = pcpt // rr
    c = lax.convert_element_type(lax.axis_index("c"), jnp.int32)
    s = lax.convert_element_type(lax.axis_index("s"), jnp.int32)
    w = c * jnp.int32(NT) + s
    def zrow(r):
        for k in range(dh // 16):
            rows[0, r, pl.ds(k * 16, 16)] = jnp.zeros((16,), jnp.float32)

    _loop(ch, zrow)
    tbase = s * jnp.int32(RPT)
    for t in range(RPT // ch):
        pltpu.sync_copy(rows.at[0],
                        acc.at[pl.ds(tbase + jnp.int32(t * ch), ch)])
    dbase = w * jnp.int32(cpt)
    plsc.subcore_barrier()

    def sidx(l):
        return sidx_all.at[pl.ds(l * jnp.int32(ch), ch)]

    def gather(l, b):
        pltpu.async_copy(ys_hbm.at[sidx(l)], rows.at[b], gsems.at[b])

    def gather_wait(l, b):
        pltpu.make_async_copy(
            ys_hbm.at[sidx(l)], rows.at[b], gsems.at[b]).wait()

    def didx_load(g, b):
        pltpu.async_copy(dst_hbm.at[dbase + g], didx_ring.at[b],
                         isems.at[b])

    def didx_wait(g, b):
        pltpu.make_async_copy(dst_hbm.at[dbase + g], didx_ring.at[b],
                              isems.at[b]).wait()

    def scatter(b):
        pltpu.async_copy(rows.at[b], acc.at[didx_ring.at[b]], ssems.at[b],
                         add=True)

    def scatter_wait(b):
        pltpu.make_async_copy(
            rows.at[b], acc.at[didx_ring.at[b]], ssems.at[b]).wait()

    for p in range(phases):
        pltpu.sync_copy(
            src_hbm.at[w * jnp.int32(phases) + jnp.int32(p)], sidx_all)
        pbase = jnp.int32(p * pcpt)

        for b in range(rr):
            didx_load(pbase + jnp.int32(b), b)
            gather(jnp.int32(b), b)

        def grp(t):
            jb = t * jnp.int32(rr)
            for b in range(rr):
                gather_wait(jb + jnp.int32(b), b)
                didx_wait(pbase + jb + jnp.int32(b), b)
                scatter(b)
            for b in range(rr):
                scatter_wait(b)
                didx_load(pbase + jb + jnp.int32(rr + b), b)
                gather(jb + jnp.int32(rr + b), b)

        _loop(ngrp - 1, grp)

        jb = jnp.int32((ngrp - 1) * rr)
        for b in range(rr):
            gather_wait(jb + jnp.int32(b), b)
            didx_wait(pbase + jb + jnp.int32(b), b)
            scatter(b)
        for b in range(rr):
            scatter_wait(b)

    plsc.subcore_barrier()
    obase = c * jnp.int32(NPAD) + tbase
    for t in range(RPT // ch):
        pltpu.sync_copy(acc.at[pl.ds(tbase + jnp.int32(t * ch), ch)],
                        rows.at[0])
        pltpu.sync_copy(rows.at[0],
                        out_hbm.at[pl.ds(obase + jnp.int32(t * ch), ch)])


@functools.cache
def _agg_call(dh, ch, rr, phases, tc_tiling):
    cpt = E_PAD // NW // ch
    return pl.kernel(
        functools.partial(_agg_body, dh, ch, rr, phases),
        out_type=jax.ShapeDtypeStruct((NC * NPAD, dh), jnp.float32),
        mesh=_mesh(),
        scratch_types=[
            pltpu.VMEM_SHARED((NPAD, dh), jnp.float32),
            pltpu.VMEM((cpt * ch // phases,), jnp.int32),
            pltpu.VMEM((rr, ch), jnp.int32),
            pltpu.VMEM((rr, ch, dh), jnp.float32),
            pltpu.SemaphoreType.DMA((rr,)),
            pltpu.SemaphoreType.DMA((rr,)),
            pltpu.SemaphoreType.DMA((rr,)),
        ],
        compiler_params=pltpu.CompilerParams(use_tc_tiling_on_sc=tc_tiling),
    )


def _tc1a_body(x_ref, w_ref, xw_ref):
    xw_ref[pl.ds(0, N_NODES), :] = jnp.dot(
        x_ref[...], w_ref[...], preferred_element_type=jnp.float32)


def _tc1a_call(x, w1):
    return pl.pallas_call(
        _tc1a_body,
        out_shape=jax.ShapeDtypeStruct((NPAD, D_HID), jnp.float32),
    )(x, w1)


def _tc1b_body(dis_ref, xw_ref, ys_ref):
    ys_ref[...] = xw_ref[...] * dis_ref[...]


def _tc1b_call(dis_rep, xw):
    return pl.pallas_call(
        _tc1b_body,
        out_shape=jax.ShapeDtypeStruct((NPAD, D_HID), jnp.float32),
    )(dis_rep, xw)


def _tc2_body(agg_ref, ys1_ref, dis_ref, b1_ref, w2_ref, ys2_ref):
    dis = dis_ref[...]
    h = jnp.maximum(dis * (agg_ref[0] + agg_ref[1] + ys1_ref[...])
                    + b1_ref[...], 0.0)
    ys2_ref[...] = jnp.dot(h, w2_ref[...],
                           preferred_element_type=jnp.float32) * dis[:, :N_CLS]


def _tc2_call(agg1, ys1, dis_rep, b1r, w2):
    return pl.pallas_call(
        _tc2_body,
        out_shape=jax.ShapeDtypeStruct((NPAD, N_CLS), jnp.float32),
    )(agg1, ys1, dis_rep, b1r, w2)


def _tc3_body(agg_ref, ys2_ref, dis_ref, b2_ref, out_ref):
    u = dis_ref[...][:N_NODES, :N_CLS] * (
        agg_ref[0][:N_NODES] + agg_ref[1][:N_NODES] + ys2_ref[...][:N_NODES])
    out_ref[...] = u + b2_ref[...]


def _tc3_call(agg2, ys2, dis_rep, b2r):
    return pl.pallas_call(
        _tc3_body,
        out_shape=jax.ShapeDtypeStruct((N_NODES, N_CLS), jnp.float32),
    )(agg2, ys2, dis_rep, b2r)


def kernel(x, edge_index, W1, b1, W2, b2):
    with jax.enable_x64(False):
        return _kernel_32(x, edge_index, W1, b1, W2, b2)


def _kernel_32(x, edge_index, W1, b1, W2, b2):
    src = edge_index[0].astype(jnp.int32)
    dst = edge_index[1].astype(jnp.int32)
    x = x.astype(jnp.float32)
    W1 = W1.astype(jnp.float32)
    W2 = W2.astype(jnp.float32)
    b1r = b1.astype(jnp.float32).reshape(1, D_HID)
    b2r = b2.astype(jnp.float32).reshape(1, N_CLS)

    pad_idx = N_NODES + jnp.arange(E_PAD - E_TOT, dtype=jnp.int32) % (
        NPAD - N_NODES)
    srcp = jnp.concatenate([src, pad_idx])
    dstp = jnp.concatenate([dst, pad_idx])
    src2 = srcp.reshape(NW, CPT * CH)
    dst2 = dstp.reshape(NW * CPT, CH)

    dst2b = dstp.reshape(NW * (E_PAD // NW // 128), 128)

    dis_rep = _dis_call()(dst2b)
    xw1 = _tc1a_call(x, W1)
    ys1 = _tc1b_call(dis_rep, xw1)
    src2h = srcp.reshape(NW * 2, CPT * CH // 2)
    agg1 = _agg_call(D_HID, CH, 5, 2, True)(src2h, dst2, ys1)
    ys2 = _tc2_call(agg1.reshape(2, NPAD, D_HID), ys1, dis_rep, b1r, W2)
    agg2 = _agg_call(N_CLS, 128, 5, 1, False)(src2, dst2b, ys2)
    return _tc3_call(agg2.reshape(2, NPAD, N_CLS), ys2, dis_rep, b2r)
